# baseline (speedup 1.0000x reference)
import functools

import jax
import jax.numpy as jnp
from jax import lax
from jax.experimental import pallas as pl
from jax.experimental.pallas import tpu as pltpu
from jax.experimental.pallas import tpu_sc as plsc

_N = 100000
_NC = 2
_NS = 16
_NW = _NC * _NS
_LANES = 16
_FULL = 3136
_LAST = _N - (_NW - 1) * _FULL
_T = 16


@functools.cache
def _build():
    @functools.partial(
        pl.kernel,
        mesh=plsc.VectorSubcoreMesh(core_axis_name="c", subcore_axis_name="s"),
        out_type=jax.ShapeDtypeStruct((_N,), jnp.float32),
        scratch_types=[
            pltpu.VMEM((_FULL,), jnp.float32),
            pltpu.VMEM((_FULL,), jnp.int32),
            pltpu.VMEM((_FULL,), jnp.float32),
            pltpu.VMEM((_T,), jnp.float32),
            pltpu.VMEM((_T,), jnp.float32),
            pltpu.SemaphoreType.DMA,
        ],
    )
    def _shift_scale(x_hbm, t_hbm, scale_hbm, shift_hbm, out_hbm,
                     x_v, t_v, o_v, scale_v, shift_v, sem):
        wid = lax.axis_index("s") * _NC + lax.axis_index("c")
        base = wid * _FULL

        def do_chunk(n_elems):
            sl_h = pl.ds(base, n_elems)
            sl_v = pl.ds(0, n_elems)
            c1 = pltpu.async_copy(x_hbm.at[sl_h], x_v.at[sl_v], sem)
            c2 = pltpu.async_copy(t_hbm.at[sl_h], t_v.at[sl_v], sem)
            c3 = pltpu.async_copy(scale_hbm, scale_v, sem)
            c4 = pltpu.async_copy(shift_hbm, shift_v, sem)
            c1.wait()
            c2.wait()
            c3.wait()
            c4.wait()
            scale_vec = scale_v[...]
            shift_vec = shift_v[...]

            nv = n_elems // _LANES
            nv_main = (nv // 8) * 8

            def step(i):
                sl = pl.ds(i * _LANES, _LANES)
                t = t_v[sl]
                s = scale_vec.at[t].get(mode="promise_in_bounds")
                h = shift_vec.at[t].get(mode="promise_in_bounds")
                o_v[sl] = s * x_v[sl] + h

            plsc.parallel_loop(0, nv_main, unroll=8)(step)
            for i in range(nv_main, nv):
                step(i)

            pltpu.sync_copy(o_v.at[sl_v], out_hbm.at[sl_h])

        @pl.when(wid < _NW - 1)
        def _():
            do_chunk(_FULL)

        @pl.when(wid == _NW - 1)
        def _():
            do_chunk(_LAST)

    return _shift_scale


def kernel(x, atom_type, scale, shift):
    return _build()(x, atom_type.astype(jnp.int32), scale, shift)


# speedup vs baseline: 1.0262x; 1.0262x over previous
import functools

import jax
import jax.numpy as jnp
from jax import lax
from jax.experimental import pallas as pl
from jax.experimental.pallas import tpu as pltpu
from jax.experimental.pallas import tpu_sc as plsc

_N = 100000
_NC = 2
_NS = 16
_NW = _NC * _NS
_LANES = 16
_FULL = 3136
_LAST = _N - (_NW - 1) * _FULL
_T = 16


@functools.cache
def _build():
    @functools.partial(
        pl.kernel,
        mesh=plsc.VectorSubcoreMesh(core_axis_name="c", subcore_axis_name="s"),
        out_type=jax.ShapeDtypeStruct((_N,), jnp.float32),
        scratch_types=[
            pltpu.VMEM((_FULL,), jnp.float32),
            pltpu.VMEM((_FULL,), jnp.int32),
            pltpu.VMEM((_FULL,), jnp.float32),
            pltpu.VMEM((_T,), jnp.float32),
            pltpu.VMEM((_T,), jnp.float32),
            pltpu.SemaphoreType.DMA,
        ],
    )
    def _shift_scale(x_hbm, t_hbm, scale_hbm, shift_hbm, out_hbm,
                     x_v, t_v, o_v, scale_v, shift_v, sem):
        wid = lax.axis_index("s") * _NC + lax.axis_index("c")
        base = jnp.minimum(wid * _FULL, _N - _FULL)

        sl_h = pl.ds(base, _FULL)
        c1 = pltpu.async_copy(x_hbm.at[sl_h], x_v, sem)
        c2 = pltpu.async_copy(t_hbm.at[sl_h], t_v, sem)
        c3 = pltpu.async_copy(scale_hbm, scale_v, sem)
        c4 = pltpu.async_copy(shift_hbm, shift_v, sem)
        c1.wait()
        c2.wait()
        c3.wait()
        c4.wait()
        scale_vec = scale_v[...]
        shift_vec = shift_v[...]

        @plsc.parallel_loop(0, _FULL // _LANES, unroll=4)
        def step(i):
            sl = pl.ds(i * _LANES, _LANES)
            t = t_v[sl]
            s = scale_vec.at[t].get(mode="promise_in_bounds")
            h = shift_vec.at[t].get(mode="promise_in_bounds")
            o_v[sl] = s * x_v[sl] + h

        pltpu.sync_copy(o_v, out_hbm.at[sl_h])

    return _shift_scale


def kernel(x, atom_type, scale, shift):
    return _build()(x, atom_type.astype(jnp.int32), scale, shift)
